# Initial kernel scaffold; baseline (speedup 1.0000x reference)
import jax
import jax.numpy as jnp
from jax.experimental import pallas as pl


def kernel(input_tensor, slices):
    raise NotImplementedError("write your pallas kernel here")


# speedup vs baseline: 1.0421x; 1.0421x over previous
import functools

import jax
import jax.numpy as jnp
from jax import lax
from jax.experimental import pallas as pl
from jax.experimental.pallas import tpu as pltpu, tpu_sc as plsc

_ROWS = 16384
_IN_COLS = 6400
_NUM_SLICES = 50
_WIDTH = 64
_OUT_COLS = _NUM_SLICES * _WIDTH

_NUM_TILES = 32
_ROWS_PER_TILE = _ROWS // _NUM_TILES
_LANES = 16


def _body(in_hbm, starts_hbm, out_hbm, sl_v, buf0, buf1, gsem0, gsem1,
          wsem0, wsem1):
    wid = lax.axis_index("s") * 2 + lax.axis_index("c")
    r0 = wid * _ROWS_PER_TILE

    pltpu.sync_copy(starts_hbm, sl_v)

    bufs = (buf0, buf1)
    gsems = (gsem0, gsem1)
    wsems = (wsem0, wsem1)

    def start_of(j):
        vec = sl_v[pl.ds((j // _LANES) * _LANES, _LANES)]
        return pl.multiple_of(vec[j % _LANES], _WIDTH)

    def gather(j):
        cst = start_of(j)
        return pltpu.make_async_copy(
            in_hbm.at[pl.ds(r0, _ROWS_PER_TILE), pl.ds(cst, _WIDTH)],
            bufs[j % 2],
            gsems[j % 2],
        )

    def writeback(j):
        return pltpu.make_async_copy(
            bufs[j % 2],
            out_hbm.at[pl.ds(r0, _ROWS_PER_TILE), pl.ds(j * _WIDTH, _WIDTH)],
            wsems[j % 2],
        )

    gather(0).start()
    for j in range(_NUM_SLICES):
        if j + 1 < _NUM_SLICES:
            if j >= 1:
                writeback(j - 1).wait()
            gather(j + 1).start()
        gather(j).wait()
        writeback(j).start()
    writeback(_NUM_SLICES - 2).wait()
    writeback(_NUM_SLICES - 1).wait()


@jax.jit
def _run(input_tensor, starts_padded):
    mesh = plsc.VectorSubcoreMesh(core_axis_name="c", subcore_axis_name="s")
    return pl.kernel(
        _body,
        out_type=jax.ShapeDtypeStruct((_ROWS, _OUT_COLS), jnp.float32),
        mesh=mesh,
        compiler_params=pltpu.CompilerParams(use_tc_tiling_on_sc=False),
        scratch_types=[
            pltpu.VMEM((64,), jnp.int32),
            pltpu.VMEM((_ROWS_PER_TILE, _WIDTH), jnp.float32),
            pltpu.VMEM((_ROWS_PER_TILE, _WIDTH), jnp.float32),
            pltpu.SemaphoreType.DMA,
            pltpu.SemaphoreType.DMA,
            pltpu.SemaphoreType.DMA,
            pltpu.SemaphoreType.DMA,
        ],
    )(input_tensor, starts_padded)


def kernel(input_tensor, slices):
    starts = slices[:, 0].astype(jnp.int32)
    starts_padded = jnp.pad(starts, (0, 64 - _NUM_SLICES))
    return _run(input_tensor, starts_padded)
